# baseline (speedup 1.0000x reference)
import jax
import jax.numpy as jnp
from jax.experimental import pallas as pl
from jax.experimental.pallas import tpu as pltpu

BN_EPS = 1e-5
K = 4
N = 2
NZ = 100


def _bn_coeffs(ys, gamma, beta, count):
    s = ys[0].sum(axis=0)
    ss = (ys[0] * ys[0]).sum(axis=0)
    for y in ys[1:]:
        s = s + y.sum(axis=0)
        ss = ss + (y * y).sum(axis=0)
    mean = s / count
    var = ss / count - mean * mean
    inv = jax.lax.rsqrt(var + BN_EPS)
    a = inv * gamma
    c = beta - mean * a
    return a, c


def _pad_input(x_ref, xp_ref, h, w, cin):
    xp_ref[:, 0:1, :, :] = jnp.zeros((N, 1, w + 2, cin), jnp.float32)
    xp_ref[:, h + 1:h + 2, :, :] = jnp.zeros((N, 1, w + 2, cin), jnp.float32)
    xp_ref[:, 1:h + 1, 0:1, :] = jnp.zeros((N, h, 1, cin), jnp.float32)
    xp_ref[:, 1:h + 1, w + 1:w + 2, :] = jnp.zeros((N, h, 1, cin),
                                                   jnp.float32)
    xp_ref[:, 1:h + 1, 1:w + 1, :] = x_ref[...]


def _up_pars(w_ref, xp_ref, h, w, cin, cout, row0=None, hs=None):
    if row0 is None:
        row0, hs = 0, h
    slabs = {}
    for ay in (0, 1, 2):
        for ax in (0, 1, 2):
            slabs[(ay, ax)] = xp_ref[:, pl.ds(row0 + ay, hs),
                                     ax:ax + w, :].reshape(N * hs * w, cin)
    pars = []
    for di in (0, 1):
        for dj in (0, 1):
            acc = None
            for ky in (di, di + 2):
                for kx in (dj, dj + 2):
                    slab = slabs[((di + ky) // 2, (dj + kx) // 2)]
                    t = ky * K + kx
                    wblk = w_ref[t * cin:(t + 1) * cin, :]
                    p = jnp.dot(slab, wblk,
                                preferred_element_type=jnp.float32)
                    acc = p if acc is None else acc + p
            pars.append((di, dj, acc))
    return pars


def _interleave(norm, h, w, c):
    r0 = jnp.stack([norm[(0, 0)], norm[(0, 1)]], axis=3).reshape(
        N, h, 2 * w, c)
    r1 = jnp.stack([norm[(1, 0)], norm[(1, 1)]], axis=3).reshape(
        N, h, 2 * w, c)
    return jnp.stack([r0, r1], axis=2).reshape(N, 2 * h, 2 * w, c)


def _l0_kernel(z_ref, w_ref, g_ref, b_ref, out_ref):
    z = z_ref[...].reshape(N, NZ)
    cout = out_ref.shape[3]
    ys = []
    for oy in range(4):
        for ox in range(4):
            t = (3 - oy) * K + (3 - ox)
            wblk = w_ref[t * NZ:(t + 1) * NZ, :]
            ys.append(jnp.dot(z, wblk, preferred_element_type=jnp.float32))
    y = jnp.stack(ys, axis=1).reshape(N * 16, cout)
    a, c = _bn_coeffs([y], g_ref[...], b_ref[...], N * 16)
    out_ref[...] = jnp.maximum(y * a + c, 0.0).reshape(N, 4, 4, cout)


def _up_kernel(x_ref, w_ref, g_ref, b_ref, out_ref, xp_ref):
    n, h, w, cin = x_ref.shape
    cout = out_ref.shape[3]
    _pad_input(x_ref, xp_ref, h, w, cin)
    pars = _up_pars(w_ref, xp_ref, h, w, cin, cout)
    a, c = _bn_coeffs([p for _, _, p in pars], g_ref[...], b_ref[...],
                      4 * N * h * w)
    norm = {(di, dj): jnp.maximum(p * a + c, 0.0).reshape(N, h, w, cout)
            for di, dj, p in pars}
    out_ref[...] = _interleave(norm, h, w, cout)


def _l4_kernel(x_ref, w_ref, out_ref, xp_ref):
    n, h, w, cin = x_ref.shape
    hs = h // 2
    row0 = pl.program_id(0) * hs
    _pad_input(x_ref, xp_ref, h, w, cin)
    pars = _up_pars(w_ref, xp_ref, h, w, cin, 3, row0=row0, hs=hs)
    t = {(di, dj): jnp.tanh(p).reshape(N, hs, w, 3) for di, dj, p in pars}
    y = _interleave(t, hs, w, 3)
    out_ref[...] = jnp.transpose(y, (0, 3, 1, 2))


@jax.jit
def _forward(z, w0, w1, w2, w3, w4, g0, b0, g1, b1, g2, b2, g3, b3):
    par = pltpu.CompilerParams(dimension_semantics=("parallel",))

    def cspec(shape, cdim):
        def imap(i):
            idx = [0] * len(shape)
            idx[cdim] = i
            return tuple(idx)
        return pl.BlockSpec(shape, imap)

    def fspec(shape):
        return pl.BlockSpec(shape, lambda i: (0,) * len(shape))

    x = pl.pallas_call(
        _l0_kernel,
        out_shape=jax.ShapeDtypeStruct((N, 4, 4, 512), jnp.float32),
        grid=(2,),
        in_specs=[fspec((N, NZ, 1, 1)), cspec((16 * NZ, 256), 1),
                  cspec((1, 256), 1), cspec((1, 256), 1)],
        out_specs=cspec((N, 4, 4, 256), 3),
        compiler_params=par,
    )(z, w0, g0, b0)

    for wm, g, b, h, cin, cout, ncore in ((w1, g1, b1, 4, 512, 256, 2),
                                          (w2, g2, b2, 8, 256, 128, 1),
                                          (w3, g3, b3, 16, 128, 64, 1)):
        ch = cout // ncore
        x = pl.pallas_call(
            _up_kernel,
            out_shape=jax.ShapeDtypeStruct((N, 2 * h, 2 * h, cout),
                                           jnp.float32),
            grid=(ncore,),
            in_specs=[fspec((N, h, h, cin)), cspec((16 * cin, ch), 1),
                      cspec((1, ch), 1), cspec((1, ch), 1)],
            out_specs=cspec((N, 2 * h, 2 * h, ch), 3),
            scratch_shapes=[pltpu.VMEM((N, h + 2, h + 2, cin), jnp.float32)],
            compiler_params=par,
        )(x, wm, g, b)

    return pl.pallas_call(
        _l4_kernel,
        out_shape=jax.ShapeDtypeStruct((N, 3, 64, 64), jnp.float32),
        grid=(2,),
        in_specs=[fspec((N, 32, 32, 64)), fspec((16 * 64, 3))],
        out_specs=cspec((N, 3, 32, 64), 2),
        scratch_shapes=[pltpu.VMEM((N, 34, 34, 64), jnp.float32)],
        compiler_params=par,
    )(x, w4)


def kernel(z, w_mat_0, w_pt_0, gamma_0, beta_0,
           w_mat_1, w_pt_1, gamma_1, beta_1,
           w_mat_2, w_pt_2, gamma_2, beta_2,
           w_mat_3, w_pt_3, gamma_3, beta_3,
           w_mat_4, w_pt_4):
    return _forward(z, w_mat_0, w_mat_1, w_mat_2, w_mat_3, w_mat_4,
                    gamma_0, beta_0, gamma_1, beta_1, gamma_2, beta_2,
                    gamma_3, beta_3)


# speedup vs baseline: 1.3610x; 1.3610x over previous
import jax
import jax.numpy as jnp
from jax.experimental import pallas as pl
from jax.experimental.pallas import tpu as pltpu

BN_EPS = 1e-5
K = 4
N = 2
NZ = 100
C0 = 512


def _bn_coeffs(ys, gamma, beta, count):
    s = ys[0].sum(axis=0)
    ss = (ys[0] * ys[0]).sum(axis=0)
    for y in ys[1:]:
        s = s + y.sum(axis=0)
        ss = ss + (y * y).sum(axis=0)
    mean = s / count
    var = ss / count - mean * mean
    inv = jax.lax.rsqrt(var + BN_EPS)
    a = inv * gamma
    c = beta - mean * a
    return a, c


def _zero_border(xp_ref, h, w, cin):
    xp_ref[:, 0:1, :, :] = jnp.zeros((N, 1, w + 2, cin), jnp.float32)
    xp_ref[:, h + 1:h + 2, :, :] = jnp.zeros((N, 1, w + 2, cin), jnp.float32)
    xp_ref[:, 1:h + 1, 0:1, :] = jnp.zeros((N, h, 1, cin), jnp.float32)
    xp_ref[:, 1:h + 1, w + 1:w + 2, :] = jnp.zeros((N, h, 1, cin),
                                                   jnp.float32)


def _up_pars(w_ref, xp_ref, h, w, cin, cout):
    slabs = {}
    for ay in (0, 1, 2):
        for ax in (0, 1, 2):
            slabs[(ay, ax)] = xp_ref[:, ay:ay + h, ax:ax + w, :].reshape(
                N * h * w, cin)
    pars = []
    for di in (0, 1):
        for dj in (0, 1):
            acc = None
            for ky in (di, di + 2):
                for kx in (dj, dj + 2):
                    slab = slabs[((di + ky) // 2, (dj + kx) // 2)]
                    t = ky * K + kx
                    wblk = w_ref[t * cin:(t + 1) * cin, :]
                    p = jnp.dot(slab, wblk,
                                preferred_element_type=jnp.float32)
                    acc = p if acc is None else acc + p
            pars.append((di, dj, acc))
    return pars


def _gen_kernel(z_ref, w0_hbm, w1_hbm, w2_hbm, w3_hbm, w4_ref,
                g0_ref, b0_ref, g1_ref, b1_ref, g2_ref, b2_ref,
                g3_ref, b3_ref, out_ref,
                w0v, w1v, w2v, w3v, xp1, xp2, xp3, xp4, sems):
    def chunked_copies(src, dst, rows, nchunk, sem_base):
        step = rows // nchunk
        return [pltpu.make_async_copy(src.at[pl.ds(i * step, step)],
                                      dst.at[pl.ds(i * step, step)],
                                      sems.at[sem_base + i])
                for i in range(nchunk)]

    cps = [chunked_copies(w0_hbm, w0v, 16 * NZ, 4, 0),
           chunked_copies(w1_hbm, w1v, 16 * 512, 8, 4),
           chunked_copies(w2_hbm, w2v, 16 * 256, 4, 12),
           chunked_copies(w3_hbm, w3v, 16 * 128, 2, 16)]
    for grp in cps:
        for cp in grp:
            cp.start()

    z = z_ref[...].reshape(N, NZ)

    for cp in cps[0]:
        cp.wait()
    ys = []
    for oy in range(4):
        for ox in range(4):
            t = (3 - oy) * K + (3 - ox)
            wblk = w0v[t * NZ:(t + 1) * NZ, :]
            ys.append(jnp.dot(z, wblk, preferred_element_type=jnp.float32))
    y = jnp.stack(ys, axis=1).reshape(N * 16, C0)
    a, c = _bn_coeffs([y], g0_ref[...], b0_ref[...], N * 16)
    y = jnp.maximum(y * a + c, 0.0)
    _zero_border(xp1, 4, 4, C0)
    xp1[:, 1:5, 1:5, :] = y.reshape(N, 4, 4, C0)

    for grp, w_ref, g_ref, b_ref, xpi, xpo, h, cin, cout in (
            (cps[1], w1v, g1_ref, b1_ref, xp1, xp2, 4, 512, 256),
            (cps[2], w2v, g2_ref, b2_ref, xp2, xp3, 8, 256, 128),
            (cps[3], w3v, g3_ref, b3_ref, xp3, xp4, 16, 128, 64)):
        for cp in grp:
            cp.wait()
        pars = _up_pars(w_ref, xpi, h, h, cin, cout)
        a, c = _bn_coeffs([p for _, _, p in pars], g_ref[...], b_ref[...],
                          4 * N * h * h)
        h2 = 2 * h
        _zero_border(xpo, h2, h2, cout)
        norm = {(di, dj): jnp.maximum(p * a + c, 0.0).reshape(N, h, h, cout)
                for di, dj, p in pars}
        r0 = jnp.stack([norm[(0, 0)], norm[(0, 1)]], axis=3).reshape(
            N, h, h2, cout)
        r1 = jnp.stack([norm[(1, 0)], norm[(1, 1)]], axis=3).reshape(
            N, h, h2, cout)
        xpo[:, 1:h2 + 1, 1:h2 + 1, :] = jnp.stack(
            [r0, r1], axis=2).reshape(N, h2, h2, cout)

    pars = _up_pars(w4_ref, xp4, 32, 32, 64, 3)
    t = {(di, dj): jnp.tanh(p).reshape(N, 32, 32, 3) for di, dj, p in pars}
    r0 = jnp.stack([t[(0, 0)], t[(0, 1)]], axis=3).reshape(N, 32, 64, 3)
    r1 = jnp.stack([t[(1, 0)], t[(1, 1)]], axis=3).reshape(N, 32, 64, 3)
    y = jnp.stack([r0, r1], axis=2).reshape(N, 64, 64, 3)
    out_ref[...] = jnp.transpose(y, (0, 3, 1, 2))


@jax.jit
def _forward(z2, w0, w1, w2, w3, w4, g0, b0, g1, b1, g2, b2, g3, b3):
    vspec = pl.BlockSpec(memory_space=pltpu.MemorySpace.VMEM)
    aspec = pl.BlockSpec(memory_space=pl.ANY)
    return pl.pallas_call(
        _gen_kernel,
        out_shape=jax.ShapeDtypeStruct((N, 3, 64, 64), jnp.float32),
        in_specs=[vspec, aspec, aspec, aspec, aspec, vspec,
                  vspec, vspec, vspec, vspec, vspec, vspec, vspec, vspec],
        out_specs=vspec,
        scratch_shapes=[
            pltpu.VMEM((16 * NZ, 512), jnp.float32),
            pltpu.VMEM((16 * 512, 256), jnp.float32),
            pltpu.VMEM((16 * 256, 128), jnp.float32),
            pltpu.VMEM((16 * 128, 64), jnp.float32),
            pltpu.VMEM((N, 6, 6, 512), jnp.float32),
            pltpu.VMEM((N, 10, 10, 256), jnp.float32),
            pltpu.VMEM((N, 18, 18, 128), jnp.float32),
            pltpu.VMEM((N, 34, 34, 64), jnp.float32),
            pltpu.SemaphoreType.DMA((18,)),
        ],
        compiler_params=pltpu.CompilerParams(
            vmem_limit_bytes=100 * 1024 * 1024),
    )(z2, w0, w1, w2, w3, w4, g0, b0, g1, b1, g2, b2, g3, b3)


def kernel(z, w_mat_0, w_pt_0, gamma_0, beta_0,
           w_mat_1, w_pt_1, gamma_1, beta_1,
           w_mat_2, w_pt_2, gamma_2, beta_2,
           w_mat_3, w_pt_3, gamma_3, beta_3,
           w_mat_4, w_pt_4):
    return _forward(z, w_mat_0, w_mat_1, w_mat_2, w_mat_3,
                    w_mat_4, gamma_0, beta_0, gamma_1, beta_1, gamma_2,
                    beta_2, gamma_3, beta_3)


# speedup vs baseline: 1.3756x; 1.0107x over previous
import jax
import jax.numpy as jnp
from jax.experimental import pallas as pl
from jax.experimental.pallas import tpu as pltpu

BN_EPS = 1e-5
K = 4
N = 2
NZ = 100
C0 = 512


def _bn_coeffs(ys, gamma, beta, count):
    s = ys[0].sum(axis=0)
    ss = (ys[0] * ys[0]).sum(axis=0)
    for y in ys[1:]:
        s = s + y.sum(axis=0)
        ss = ss + (y * y).sum(axis=0)
    mean = s / count
    var = ss / count - mean * mean
    inv = jax.lax.rsqrt(var + BN_EPS)
    a = inv * gamma
    c = beta - mean * a
    return a, c


def _zero_border(xp_ref, h, w, cin):
    xp_ref[:, 0:1, :, :] = jnp.zeros((N, 1, w + 2, cin), jnp.float32)
    xp_ref[:, h + 1:h + 2, :, :] = jnp.zeros((N, 1, w + 2, cin), jnp.float32)
    xp_ref[:, 1:h + 1, 0:1, :] = jnp.zeros((N, h, 1, cin), jnp.float32)
    xp_ref[:, 1:h + 1, w + 1:w + 2, :] = jnp.zeros((N, h, 1, cin),
                                                   jnp.float32)


def _up_pars(w_ref, xp_ref, h, w, cin, cout):
    slabs = {}
    for ay in (0, 1, 2):
        for ax in (0, 1, 2):
            slabs[(ay, ax)] = xp_ref[:, ay:ay + h, ax:ax + w, :].reshape(
                N * h * w, cin)
    pars = []
    for di in (0, 1):
        for dj in (0, 1):
            acc = None
            for ky in (di, di + 2):
                for kx in (dj, dj + 2):
                    slab = slabs[((di + ky) // 2, (dj + kx) // 2)]
                    t = ky * K + kx
                    wblk = w_ref[t * cin:(t + 1) * cin, :]
                    p = jnp.dot(slab, wblk,
                                preferred_element_type=jnp.float32)
                    acc = p if acc is None else acc + p
            pars.append((di, dj, acc))
    return pars


def _gen_kernel(z_ref, w0_hbm, w1_hbm, w2_hbm, w3_hbm, w4_ref,
                g0_ref, b0_ref, g1_ref, b1_ref, g2_ref, b2_ref,
                g3_ref, b3_ref, out_ref,
                w0v, w1v, w2v, w3v, xp1, xp2, xp3, xp4, sems):
    def chunked_copies(src, dst, rows, nchunk, sem_base):
        step = rows // nchunk
        return [pltpu.make_async_copy(src.at[pl.ds(i * step, step)],
                                      dst.at[pl.ds(i * step, step)],
                                      sems.at[sem_base + i])
                for i in range(nchunk)]

    cps = [chunked_copies(w0_hbm, w0v, 16 * NZ, 4, 0),
           chunked_copies(w1_hbm, w1v, 16 * 512, 8, 4),
           chunked_copies(w2_hbm, w2v, 16 * 256, 4, 12),
           chunked_copies(w3_hbm, w3v, 16 * 128, 2, 16)]
    for grp in cps:
        for cp in grp:
            cp.start()

    z = z_ref[...].reshape(N, NZ)

    for cp in cps[0]:
        cp.wait()
    ys = []
    for oy in range(4):
        for ox in range(4):
            t = (3 - oy) * K + (3 - ox)
            wblk = w0v[t * NZ:(t + 1) * NZ, :]
            ys.append(jnp.dot(z, wblk, preferred_element_type=jnp.float32))
    y = jnp.stack(ys, axis=1).reshape(N * 16, C0)
    a, c = _bn_coeffs([y], g0_ref[...], b0_ref[...], N * 16)
    y = jnp.maximum(y * a + c, 0.0)
    _zero_border(xp1, 4, 4, C0)
    xp1[:, 1:5, 1:5, :] = y.reshape(N, 4, 4, C0)

    for grp, w_ref, g_ref, b_ref, xpi, xpo, h, cin, cout in (
            (cps[1], w1v, g1_ref, b1_ref, xp1, xp2, 4, 512, 256),
            (cps[2], w2v, g2_ref, b2_ref, xp2, xp3, 8, 256, 128),
            (cps[3], w3v, g3_ref, b3_ref, xp3, xp4, 16, 128, 64)):
        for cp in grp:
            cp.wait()
        pars = _up_pars(w_ref, xpi, h, h, cin, cout)
        a, c = _bn_coeffs([p for _, _, p in pars], g_ref[...], b_ref[...],
                          4 * N * h * h)
        h2 = 2 * h
        _zero_border(xpo, h2, h2, cout)
        norm = {(di, dj): jnp.maximum(p * a + c, 0.0).reshape(N, h, h, cout)
                for di, dj, p in pars}
        if cout == 128:
            for (di, dj), v in norm.items():
                xpo[:, 1 + di:1 + h2:2, 1 + dj:1 + h2:2, :] = v
        else:
            r0 = jnp.stack([norm[(0, 0)], norm[(0, 1)]], axis=3).reshape(
                N, h, h2, cout)
            r1 = jnp.stack([norm[(1, 0)], norm[(1, 1)]], axis=3).reshape(
                N, h, h2, cout)
            xpo[:, 1:h2 + 1, 1:h2 + 1, :] = jnp.stack(
                [r0, r1], axis=2).reshape(N, h2, h2, cout)

    pars = _up_pars(w4_ref, xp4, 32, 32, 64, 3)
    t = {(di, dj): jnp.tanh(p).reshape(N, 32, 32, 3) for di, dj, p in pars}
    r0 = jnp.stack([t[(0, 0)], t[(0, 1)]], axis=3).reshape(N, 32, 64, 3)
    r1 = jnp.stack([t[(1, 0)], t[(1, 1)]], axis=3).reshape(N, 32, 64, 3)
    y = jnp.stack([r0, r1], axis=2).reshape(N, 64, 64, 3)
    out_ref[...] = jnp.transpose(y, (0, 3, 1, 2))


@jax.jit
def _forward(z2, w0, w1, w2, w3, w4, g0, b0, g1, b1, g2, b2, g3, b3):
    vspec = pl.BlockSpec(memory_space=pltpu.MemorySpace.VMEM)
    aspec = pl.BlockSpec(memory_space=pl.ANY)
    return pl.pallas_call(
        _gen_kernel,
        out_shape=jax.ShapeDtypeStruct((N, 3, 64, 64), jnp.float32),
        in_specs=[vspec, aspec, aspec, aspec, aspec, vspec,
                  vspec, vspec, vspec, vspec, vspec, vspec, vspec, vspec],
        out_specs=vspec,
        scratch_shapes=[
            pltpu.VMEM((16 * NZ, 512), jnp.float32),
            pltpu.VMEM((16 * 512, 256), jnp.float32),
            pltpu.VMEM((16 * 256, 128), jnp.float32),
            pltpu.VMEM((16 * 128, 64), jnp.float32),
            pltpu.VMEM((N, 6, 6, 512), jnp.float32),
            pltpu.VMEM((N, 10, 10, 256), jnp.float32),
            pltpu.VMEM((N, 18, 18, 128), jnp.float32),
            pltpu.VMEM((N, 34, 34, 64), jnp.float32),
            pltpu.SemaphoreType.DMA((18,)),
        ],
        compiler_params=pltpu.CompilerParams(
            vmem_limit_bytes=100 * 1024 * 1024),
    )(z2, w0, w1, w2, w3, w4, g0, b0, g1, b1, g2, b2, g3, b3)


def kernel(z, w_mat_0, w_pt_0, gamma_0, beta_0,
           w_mat_1, w_pt_1, gamma_1, beta_1,
           w_mat_2, w_pt_2, gamma_2, beta_2,
           w_mat_3, w_pt_3, gamma_3, beta_3,
           w_mat_4, w_pt_4):
    return _forward(z, w_mat_0, w_mat_1, w_mat_2, w_mat_3,
                    w_mat_4, gamma_0, beta_0, gamma_1, beta_1, gamma_2,
                    beta_2, gamma_3, beta_3)
